# Initial kernel scaffold; baseline (speedup 1.0000x reference)
import jax
import jax.numpy as jnp
from jax.experimental import pallas as pl


def kernel(x, edge_index, W1, b1, W2, b2, gamma, beta):
    raise NotImplementedError("write your pallas kernel here")


# speedup vs baseline: 3.3419x; 3.3419x over previous
import functools

import jax
import jax.numpy as jnp
from jax import lax
from jax.experimental import pallas as pl
from jax.experimental.pallas import tpu as pltpu
from jax.experimental.pallas import tpu_sc as plsc

N = 10000
E = 320000
D = 128

NC = 2
NS = 16
NW = NC * NS

CL = 128
EPW = 10240
CHUNKS = EPW // CL
EPAD = EPW * NW
NPAD = 10240
ZROWS = NPAD // NS

_sc_mesh = plsc.VectorSubcoreMesh(core_axis_name="c", subcore_axis_name="s")


@functools.partial(
    pl.kernel,
    out_type=jax.ShapeDtypeStruct((NC, NPAD, D), jnp.float32),
    mesh=_sc_mesh,
    scratch_types=[
        pltpu.VMEM((CHUNKS, CL), jnp.int32),
        pltpu.VMEM((CHUNKS, CL), jnp.int32),
        pltpu.VMEM((CL, D), jnp.float32),
        pltpu.VMEM_SHARED((NPAD, D), jnp.float32),
        pltpu.SemaphoreType.DMA,
    ],
)
def _sc_agg(src_hbm, dst_hbm, x_hbm, zeros_hbm, out_hbm,
            src_v, dst_v, rows_v, agg_sh, sem):
    cid = lax.axis_index("c")
    sid = lax.axis_index("s")
    wid = sid * NC + cid

    pltpu.sync_copy(zeros_hbm, agg_sh.at[pl.ds(sid * ZROWS, ZROWS)])

    pltpu.sync_copy(src_hbm.at[wid], src_v)
    pltpu.sync_copy(dst_hbm.at[wid], dst_v)
    plsc.subcore_barrier()

    def body(j, carry):
        pltpu.async_copy(x_hbm.at[src_v.at[j]], rows_v, sem).wait()
        pltpu.sync_copy(rows_v, agg_sh.at[dst_v.at[j]], add=True)
        return carry

    lax.fori_loop(0, CHUNKS, body, 0)
    plsc.subcore_barrier()

    pltpu.sync_copy(agg_sh.at[pl.ds(sid * ZROWS, ZROWS)],
                    out_hbm.at[cid, pl.ds(sid * ZROWS, ZROWS)])


def _tc_body(x_ref, p_ref, w1_ref, b1_ref, w2_ref, b2_ref, g_ref, bt_ref,
             out_ref):
    x = x_ref[...]
    h0 = x + p_ref[0] + p_ref[1]
    h1 = jnp.dot(h0, w1_ref[...], preferred_element_type=jnp.float32)
    h1 = jnp.maximum(h1 + b1_ref[...], 0.0)
    h2 = jnp.dot(h1, w2_ref[...], preferred_element_type=jnp.float32)
    h2 = h2 + b2_ref[...]
    mean = jnp.mean(h2, axis=0, keepdims=True)
    var = jnp.mean((h2 - mean) ** 2, axis=0, keepdims=True)
    out_ref[...] = ((h2 - mean) * lax.rsqrt(var + 1e-5) * g_ref[...]
                    + bt_ref[...] + x)


_tc_mlp = pl.pallas_call(
    _tc_body,
    out_shape=jax.ShapeDtypeStruct((N, D), jnp.float32),
)


def kernel(x, edge_index, W1, b1, W2, b2, gamma, beta):
    src = edge_index[0].astype(jnp.int32)
    dst = edge_index[1].astype(jnp.int32)
    pad = EPAD - E
    src_p = jnp.concatenate([src, jnp.zeros((pad,), jnp.int32)])
    dst_p = jnp.concatenate(
        [dst, N + (jnp.arange(pad, dtype=jnp.int32) % (NPAD - N))])
    src_p = src_p.reshape(NW, CHUNKS, CL)
    dst_p = dst_p.reshape(NW, CHUNKS, CL)
    zeros = jnp.zeros((ZROWS, D), jnp.float32)

    partial = _sc_agg(src_p, dst_p, x, zeros)[:, :N, :]
    return _tc_mlp(x, partial, W1, b1.reshape(1, D), W2, b2.reshape(1, D),
                   gamma.reshape(1, D), beta.reshape(1, D))


# speedup vs baseline: 3.7182x; 1.1126x over previous
import functools

import jax
import jax.numpy as jnp
from jax import lax
from jax.experimental import pallas as pl
from jax.experimental.pallas import tpu as pltpu
from jax.experimental.pallas import tpu_sc as plsc

N = 10000
E = 320000
D = 128

NC = 2
NS = 16
NW = NC * NS

CL = 128
EPW = 10240
CHUNKS = EPW // CL
EPAD = EPW * NW
NPAD = 10240
ZROWS = NPAD // NS

_sc_mesh = plsc.VectorSubcoreMesh(core_axis_name="c", subcore_axis_name="s")


@functools.partial(
    pl.kernel,
    out_type=jax.ShapeDtypeStruct((NC, NPAD, D), jnp.float32),
    mesh=_sc_mesh,
    scratch_types=[
        pltpu.VMEM((CHUNKS // 2, CL), jnp.int32),
        pltpu.VMEM((CHUNKS // 2, CL), jnp.int32),
        pltpu.VMEM((CL, D), jnp.float32),
        pltpu.VMEM((CL, D), jnp.float32),
        pltpu.VMEM_SHARED((NPAD, D), jnp.float32),
        pltpu.SemaphoreType.DMA,
        pltpu.SemaphoreType.DMA,
    ],
)
def _sc_agg(src_hbm, dst_hbm, x_hbm, zeros_hbm, out_hbm,
            src_v, dst_v, rows0_v, rows1_v, agg_sh, sem0, sem1):
    cid = lax.axis_index("c")
    sid = lax.axis_index("s")
    wid = sid * NC + cid

    pltpu.sync_copy(zeros_hbm, agg_sh.at[pl.ds(sid * ZROWS, ZROWS)])
    plsc.subcore_barrier()

    bufs = (rows0_v, rows1_v)
    sems = (sem0, sem1)
    HC = CHUNKS // 2

    for h in range(2):
        pltpu.sync_copy(src_hbm.at[wid, pl.ds(h * HC, HC)], src_v)
        pltpu.sync_copy(dst_hbm.at[wid, pl.ds(h * HC, HC)], dst_v)
        pltpu.async_copy(x_hbm.at[src_v.at[0]], rows0_v, sem0)

        def body(i, carry):
            j0 = 2 * i
            for k in range(2):
                j = j0 + k
                buf, sem = bufs[k], sems[k]
                nbuf, nsem = bufs[1 - k], sems[1 - k]

                @pl.when(j + 1 < HC)
                def _():
                    pltpu.async_copy(x_hbm.at[src_v.at[j + 1]], nbuf, nsem)

                pltpu.make_async_copy(x_hbm.at[src_v.at[j]], buf, sem).wait()
                pltpu.sync_copy(buf, agg_sh.at[dst_v.at[j]], add=True)
            return carry

        lax.fori_loop(0, HC // 2, body, 0)

    plsc.subcore_barrier()

    pltpu.sync_copy(agg_sh.at[pl.ds(sid * ZROWS, ZROWS)],
                    out_hbm.at[cid, pl.ds(sid * ZROWS, ZROWS)])


def _tc_body(x_ref, p_ref, w1_ref, b1_ref, w2_ref, b2_ref, g_ref, bt_ref,
             out_ref):
    x = x_ref[...]
    h0 = x + p_ref[0, :N] + p_ref[1, :N]
    h1 = jnp.dot(h0, w1_ref[...], preferred_element_type=jnp.float32)
    h1 = jnp.maximum(h1 + b1_ref[...], 0.0)
    h2 = jnp.dot(h1, w2_ref[...], preferred_element_type=jnp.float32)
    h2 = h2 + b2_ref[...]
    mean = jnp.mean(h2, axis=0, keepdims=True)
    var = jnp.mean((h2 - mean) ** 2, axis=0, keepdims=True)
    out_ref[...] = ((h2 - mean) * lax.rsqrt(var + 1e-5) * g_ref[...]
                    + bt_ref[...] + x)


_tc_mlp = pl.pallas_call(
    _tc_body,
    out_shape=jax.ShapeDtypeStruct((N, D), jnp.float32),
)


def kernel(x, edge_index, W1, b1, W2, b2, gamma, beta):
    src = edge_index[0].astype(jnp.int32)
    dst = edge_index[1].astype(jnp.int32)
    pad = EPAD - E
    src_p = jnp.concatenate([src, jnp.zeros((pad,), jnp.int32)])
    dst_p = jnp.concatenate(
        [dst, N + (jnp.arange(pad, dtype=jnp.int32) % (NPAD - N))])
    src_p = src_p.reshape(NW, CHUNKS, CL)
    dst_p = dst_p.reshape(NW, CHUNKS, CL)
    zeros = jnp.zeros((ZROWS, D), jnp.float32)

    partial = _sc_agg(src_p, dst_p, x, zeros)
    return _tc_mlp(x, partial, W1, b1.reshape(1, D), W2, b2.reshape(1, D),
                   gamma.reshape(1, D), beta.reshape(1, D))


# speedup vs baseline: 12.4675x; 3.3531x over previous
import functools

import jax
import jax.numpy as jnp
from jax import lax
from jax.experimental import pallas as pl
from jax.experimental.pallas import tpu as pltpu
from jax.experimental.pallas import tpu_sc as plsc

N = 10000
E = 320000
D = 128

NC = 2
NS = 16
NW = NC * NS

CL = 128
EPW = 10240
CHUNKS = EPW // CL
EPAD = EPW * NW
NPAD = 10240
ZROWS = NPAD // NS

_sc_mesh = plsc.VectorSubcoreMesh(core_axis_name="c", subcore_axis_name="s")


@functools.partial(
    pl.kernel,
    out_type=jax.ShapeDtypeStruct((NC, NPAD, D), jnp.float32),
    mesh=_sc_mesh,
    scratch_types=[
        pltpu.VMEM((CHUNKS // 2, CL), jnp.int32),
        pltpu.VMEM((CHUNKS // 2, CL), jnp.int32),
        pltpu.VMEM((CL, D), jnp.float32),
        pltpu.VMEM((CL, D), jnp.float32),
        pltpu.VMEM_SHARED((NPAD, D), jnp.float32),
        pltpu.SemaphoreType.DMA,
        pltpu.SemaphoreType.DMA,
    ],
)
def _sc_agg(src_hbm, dst_hbm, x_hbm, zeros_hbm, out_hbm,
            src_v, dst_v, rows0_v, rows1_v, agg_sh, sem0, sem1):
    cid = lax.axis_index("c")
    sid = lax.axis_index("s")
    wid = sid * NC + cid

    pltpu.sync_copy(zeros_hbm, agg_sh.at[pl.ds(sid * ZROWS, ZROWS)])
    plsc.subcore_barrier()

    bufs = (rows0_v, rows1_v)
    sems = (sem0, sem1)
    HC = CHUNKS // 2

    for h in range(2):
        pltpu.sync_copy(src_hbm.at[wid, pl.ds(h * HC, HC)], src_v)
        pltpu.sync_copy(dst_hbm.at[wid, pl.ds(h * HC, HC)], dst_v)
        pltpu.async_copy(x_hbm.at[src_v.at[0]], rows0_v, sem0)

        def body(i, carry):
            j0 = 2 * i
            for k in range(2):
                j = j0 + k
                buf, sem = bufs[k], sems[k]
                nbuf, nsem = bufs[1 - k], sems[1 - k]

                @pl.when(j + 1 < HC)
                def _():
                    pltpu.async_copy(x_hbm.at[src_v.at[j + 1]], nbuf, nsem)

                pltpu.make_async_copy(x_hbm.at[src_v.at[j]], buf, sem).wait()
                pltpu.sync_copy(buf, agg_sh.at[dst_v.at[j]], add=True)
            return carry

        lax.fori_loop(0, HC // 2, body, 0)

    plsc.subcore_barrier()

    pltpu.sync_copy(agg_sh.at[pl.ds(sid * ZROWS, ZROWS)],
                    out_hbm.at[cid, pl.ds(sid * ZROWS, ZROWS)])


def _tc_body(x_ref, p_ref, w1_ref, b1_ref, w2_ref, b2_ref, g_ref, bt_ref,
             out_ref):
    x = x_ref[...]
    h0 = x + p_ref[0, :N] + p_ref[1, :N]
    h1 = jnp.dot(h0, w1_ref[...], preferred_element_type=jnp.float32)
    h1 = jnp.maximum(h1 + b1_ref[...], 0.0)
    h2 = jnp.dot(h1, w2_ref[...], preferred_element_type=jnp.float32)
    h2 = h2 + b2_ref[...]
    mean = jnp.mean(h2, axis=0, keepdims=True)
    var = jnp.mean((h2 - mean) ** 2, axis=0, keepdims=True)
    out_ref[...] = ((h2 - mean) * lax.rsqrt(var + 1e-5) * g_ref[...]
                    + bt_ref[...] + x)


_tc_mlp = pl.pallas_call(
    _tc_body,
    out_shape=jax.ShapeDtypeStruct((N, D), jnp.float32),
)


def kernel(x, edge_index, W1, b1, W2, b2, gamma, beta):
    src = edge_index[0].astype(jnp.int32)
    dst = edge_index[1].astype(jnp.int32)
    pad = EPAD - E
    src_p = jnp.concatenate([src, jnp.arange(pad, dtype=jnp.int32) % N])
    dst_p = jnp.concatenate(
        [dst, N + (jnp.arange(pad, dtype=jnp.int32) % (NPAD - N))])
    src_p = src_p.reshape(NW, CHUNKS, CL)
    dst_p = dst_p.reshape(NW, CHUNKS, CL)
    zeros = jnp.zeros((ZROWS, D), jnp.float32)

    partial = _sc_agg(src_p, dst_p, x, zeros)
    return _tc_mlp(x, partial, W1, b1.reshape(1, D), W2, b2.reshape(1, D),
                   gamma.reshape(1, D), beta.reshape(1, D))


# speedup vs baseline: 12.7625x; 1.0237x over previous
import functools

import jax
import jax.numpy as jnp
from jax import lax
from jax.experimental import pallas as pl
from jax.experimental.pallas import tpu as pltpu
from jax.experimental.pallas import tpu_sc as plsc

N = 10000
E = 320000
D = 128

NC = 2
NS = 16
NW = NC * NS

CL = 64
EPW = 10240
CHUNKS = EPW // CL
EPAD = EPW * NW
NB = 4
NSTAGE = 4
HC = CHUNKS // NSTAGE
NPAD = 10240
ZROWS = NPAD // NS

_sc_mesh = plsc.VectorSubcoreMesh(core_axis_name="c", subcore_axis_name="s")


@functools.partial(
    pl.kernel,
    out_type=jax.ShapeDtypeStruct((NC, NPAD, D), jnp.float32),
    mesh=_sc_mesh,
    scratch_types=[
        pltpu.VMEM((HC, CL), jnp.int32),
        pltpu.VMEM((HC, CL), jnp.int32),
        [pltpu.VMEM((CL, D), jnp.float32)] * NB,
        [pltpu.SemaphoreType.DMA] * NB,
        [pltpu.SemaphoreType.DMA] * NB,
        pltpu.VMEM_SHARED((NPAD, D), jnp.float32),
    ],
)
def _sc_agg(src_hbm, dst_hbm, x_hbm, zeros_hbm, out_hbm,
            src_v, dst_v, bufs, gsems, ssems, agg_sh):
    cid = lax.axis_index("c")
    sid = lax.axis_index("s")
    wid = sid * NC + cid

    pltpu.sync_copy(zeros_hbm, agg_sh.at[pl.ds(sid * ZROWS, ZROWS)])
    plsc.subcore_barrier()

    for h in range(NSTAGE):
        pltpu.sync_copy(src_hbm.at[wid, pl.ds(h * HC, HC)], src_v)
        pltpu.sync_copy(dst_hbm.at[wid, pl.ds(h * HC, HC)], dst_v)
        for k in range(NB - 1):
            pltpu.async_copy(x_hbm.at[src_v.at[k]], bufs[k], gsems[k])

        def body(i, carry):
            j0 = NB * i
            for k in range(NB):
                j = j0 + k
                kf = (k + NB - 1) % NB

                @pl.when(j + NB - 1 < HC)
                def _():
                    @pl.when(j >= 1)
                    def _():
                        pltpu.make_async_copy(
                            bufs[kf], agg_sh.at[dst_v.at[0]],
                            ssems[kf]).wait()
                    pltpu.async_copy(x_hbm.at[src_v.at[j + NB - 1]],
                                     bufs[kf], gsems[kf])

                pltpu.make_async_copy(x_hbm.at[src_v.at[j]], bufs[k],
                                      gsems[k]).wait()
                pltpu.async_copy(bufs[k], agg_sh.at[dst_v.at[j]], ssems[k],
                                 add=True)
            return carry

        lax.fori_loop(0, HC // NB, body, 0)
        for k in range(NB):
            pltpu.make_async_copy(bufs[k], agg_sh.at[dst_v.at[0]],
                                  ssems[k]).wait()

    plsc.subcore_barrier()

    pltpu.sync_copy(agg_sh.at[pl.ds(sid * ZROWS, ZROWS)],
                    out_hbm.at[cid, pl.ds(sid * ZROWS, ZROWS)])


def _tc_body(x_ref, p_ref, w1_ref, b1_ref, w2_ref, b2_ref, g_ref, bt_ref,
             out_ref):
    x = x_ref[...]
    h0 = x + p_ref[0, :N] + p_ref[1, :N]
    h1 = jnp.dot(h0, w1_ref[...], preferred_element_type=jnp.float32)
    h1 = jnp.maximum(h1 + b1_ref[...], 0.0)
    h2 = jnp.dot(h1, w2_ref[...], preferred_element_type=jnp.float32)
    h2 = h2 + b2_ref[...]
    mean = jnp.mean(h2, axis=0, keepdims=True)
    var = jnp.mean((h2 - mean) ** 2, axis=0, keepdims=True)
    out_ref[...] = ((h2 - mean) * lax.rsqrt(var + 1e-5) * g_ref[...]
                    + bt_ref[...] + x)


_tc_mlp = pl.pallas_call(
    _tc_body,
    out_shape=jax.ShapeDtypeStruct((N, D), jnp.float32),
)


def kernel(x, edge_index, W1, b1, W2, b2, gamma, beta):
    src = edge_index[0].astype(jnp.int32)
    dst = edge_index[1].astype(jnp.int32)
    pad = EPAD - E
    src_p = jnp.concatenate([src, jnp.arange(pad, dtype=jnp.int32) % N])
    dst_p = jnp.concatenate(
        [dst, N + (jnp.arange(pad, dtype=jnp.int32) % (NPAD - N))])
    src_p = src_p.reshape(NW, CHUNKS, CL)
    dst_p = dst_p.reshape(NW, CHUNKS, CL)
    zeros = jnp.zeros((ZROWS, D), jnp.float32)

    partial = _sc_agg(src_p, dst_p, x, zeros)
    return _tc_mlp(x, partial, W1, b1.reshape(1, D), W2, b2.reshape(1, D),
                   gamma.reshape(1, D), beta.reshape(1, D))


# speedup vs baseline: 13.1659x; 1.0316x over previous
import functools

import jax
import jax.numpy as jnp
import numpy as np
from jax import lax
from jax.experimental import pallas as pl
from jax.experimental.pallas import tpu as pltpu
from jax.experimental.pallas import tpu_sc as plsc

N = 10000
E = 320000
D = 128

NC = 2
NS = 16
NW = NC * NS

CL = 64
EPW = 10240
CHUNKS = EPW // CL
EPAD = EPW * NW
NB = 4
NSTAGE = 4
HC = CHUNKS // NSTAGE
NPAD = 10240
ZROWS = NPAD // NS

_PAD = EPAD - E
_SRC_PAD = np.arange(_PAD, dtype=np.int32) % N
_DST_PAD = (N + np.arange(_PAD, dtype=np.int32) % (NPAD - N)).astype(np.int32)

_sc_mesh = plsc.VectorSubcoreMesh(core_axis_name="c", subcore_axis_name="s")


@functools.partial(
    pl.kernel,
    out_type=jax.ShapeDtypeStruct((NC, NPAD, D), jnp.float32),
    mesh=_sc_mesh,
    scratch_types=[
        pltpu.VMEM((HC * CL,), jnp.int32),
        pltpu.VMEM((HC, CL), jnp.int32),
        [pltpu.VMEM((CL, D), jnp.float32)] * NB,
        [pltpu.SemaphoreType.DMA] * NB,
        [pltpu.SemaphoreType.DMA] * NB,
        pltpu.VMEM_SHARED((NPAD, D), jnp.float32),
    ],
)
def _sc_agg(src_hbm, dst_hbm, x_hbm, out_hbm,
            src_v, dst_v, bufs, gsems, ssems, agg_sh):
    cid = lax.axis_index("c")
    sid = lax.axis_index("s")
    wid = sid * NC + cid

    def zbody(i, carry):
        bufs[0][i // 8, pl.ds((i % 8) * 16, 16)] = jnp.zeros(
            (16,), jnp.float32)
        return carry

    lax.fori_loop(0, CL * 8, zbody, 0)
    for t in range(ZROWS // CL):
        pltpu.sync_copy(bufs[0], agg_sh.at[pl.ds(sid * ZROWS + t * CL, CL)])
    plsc.subcore_barrier()

    for h in range(NSTAGE):
        pltpu.sync_copy(
            src_hbm.at[pl.ds(wid * EPW + h * HC * CL, HC * CL)], src_v)
        pltpu.sync_copy(dst_hbm.at[wid, pl.ds(h * HC, HC)], dst_v)
        for k in range(NB - 1):
            pltpu.async_copy(x_hbm.at[src_v.at[pl.ds(k * CL, CL)]],
                             bufs[k], gsems[k])

        def body(i, carry):
            j0 = NB * i
            for k in range(NB):
                j = j0 + k
                kf = (k + NB - 1) % NB

                @pl.when(j + NB - 1 < HC)
                def _():
                    @pl.when(j >= 1)
                    def _():
                        pltpu.make_async_copy(
                            bufs[kf], agg_sh.at[dst_v.at[0]],
                            ssems[kf]).wait()
                    pltpu.async_copy(
                        x_hbm.at[src_v.at[pl.ds((j + NB - 1) * CL, CL)]],
                        bufs[kf], gsems[kf])

                pltpu.make_async_copy(
                    x_hbm.at[src_v.at[pl.ds(j * CL, CL)]], bufs[k],
                    gsems[k]).wait()
                pltpu.async_copy(bufs[k], agg_sh.at[dst_v.at[j]], ssems[k],
                                 add=True)
            return carry

        lax.fori_loop(0, HC // NB, body, 0)
        for k in range(NB):
            pltpu.make_async_copy(bufs[k], agg_sh.at[dst_v.at[0]],
                                  ssems[k]).wait()

    plsc.subcore_barrier()

    pltpu.sync_copy(agg_sh.at[pl.ds(sid * ZROWS, ZROWS)],
                    out_hbm.at[cid, pl.ds(sid * ZROWS, ZROWS)])


def _tc_body(x_ref, p_ref, w1_ref, b1_ref, w2_ref, b2_ref, g_ref, bt_ref,
             out_ref):
    x = x_ref[...]
    h0 = x + p_ref[0, :N] + p_ref[1, :N]
    h1 = jnp.dot(h0, w1_ref[...], preferred_element_type=jnp.float32)
    h1 = jnp.maximum(h1 + b1_ref[...], 0.0)
    h2 = jnp.dot(h1, w2_ref[...], preferred_element_type=jnp.float32)
    h2 = h2 + b2_ref[...]
    mean = jnp.mean(h2, axis=0, keepdims=True)
    var = jnp.mean((h2 - mean) ** 2, axis=0, keepdims=True)
    out_ref[...] = ((h2 - mean) * lax.rsqrt(var + 1e-5) * g_ref[...]
                    + bt_ref[...] + x)


_tc_mlp = pl.pallas_call(
    _tc_body,
    out_shape=jax.ShapeDtypeStruct((N, D), jnp.float32),
)


def kernel(x, edge_index, W1, b1, W2, b2, gamma, beta):
    src = edge_index[0].astype(jnp.int32)
    dst = edge_index[1].astype(jnp.int32)
    src_p = jnp.concatenate([src, jnp.asarray(_SRC_PAD)])
    dst_p = jnp.concatenate([dst, jnp.asarray(_DST_PAD)]).reshape(
        NW, CHUNKS, CL)

    partial = _sc_agg(src_p, dst_p, x)
    return _tc_mlp(x, partial, W1, b1.reshape(1, D), W2, b2.reshape(1, D),
                   gamma.reshape(1, D), beta.reshape(1, D))
